# baseline (speedup 1.0000x reference)
import jax
import jax.numpy as jnp
import numpy as np
from jax import lax
from jax.experimental import pallas as pl
from jax.experimental.pallas import tpu as pltpu
from jax.experimental.pallas import tpu_sc as plsc

_NC = 2
_NS = 16
_NW = _NC * _NS
_C = 80
_T = 126
_ZR = 128
_NPAD = 10240


def _spmm_body(src_hbm, dst_hbm, w_hbm, x_hbm, out_hbm,
               acc,
               srcv0, srcv1, srcv2, dstv0, dstv1, dstv2,
               dstS0, dstS1, dstS2, wv0, wv1, wv2,
               rows0, rows1, rows2, zero_v,
               sg0, sg1, sg2, ss0, ss1, ss2, si0, si1, si2):
    n = acc.shape[0]
    epw = _T * _C
    c = lax.axis_index("c")
    s = lax.axis_index("s")
    wid = s * _NC + c
    rps = n // _NS

    srcv = [srcv0, srcv1, srcv2]
    dstv = [dstv0, dstv1, dstv2]
    dstS = [dstS0, dstS1, dstS2]
    wv = [wv0, wv1, wv2]
    rows = [rows0, rows1, rows2]
    sg = [sg0, sg1, sg2]
    ss = [ss0, ss1, ss2]
    si = [si0, si1, si2]

    def issue_idx(cnum, b):
        base = wid * epw + cnum * _C
        pltpu.async_copy(src_hbm.at[pl.ds(base, _C)], srcv[b], si[b])
        pltpu.async_copy(dst_hbm.at[pl.ds(base, _C)], dstv[b], si[b])
        pltpu.async_copy(w_hbm.at[pl.ds(base, _C)], wv[b], si[b])

    def wait_idx(b):
        pltpu.make_async_copy(src_hbm.at[pl.ds(0, _C)], srcv[b], si[b]).wait()
        pltpu.make_async_copy(dst_hbm.at[pl.ds(0, _C)], dstv[b], si[b]).wait()
        pltpu.make_async_copy(w_hbm.at[pl.ds(0, _C)], wv[b], si[b]).wait()

    def issue_gather(b):
        pltpu.async_copy(x_hbm.at[srcv[b]], rows[b], sg[b])

    def wait_gather(b):
        pltpu.make_async_copy(x_hbm.at[srcv[b]], rows[b], sg[b]).wait()

    def issue_scatter(b):
        pltpu.async_copy(rows[b], acc.at[dstS[b]], ss[b], add=True)

    def wait_scatter(b):
        pltpu.make_async_copy(rows[b], acc.at[dstS[b]], ss[b]).wait()

    def multiply(b):
        def gloop(g, cc):
            w16 = wv[b][pl.ds(g * 16, 16)]
            for rr in range(16):
                wsp = jnp.broadcast_to(w16[rr], (16,))
                for j in range(8):
                    vec = rows[b][g * 16 + rr, pl.ds(j * 16, 16)]
                    rows[b][g * 16 + rr, pl.ds(j * 16, 16)] = vec * wsp
            return cc
        lax.fori_loop(0, _C // 16, gloop, 0)

    def do_chunk(cnum, b, issue_idx_next=True, issue_gather_next=True,
                 skip_ss_wait=False):
        wait_gather(b)
        multiply(b)
        for i in range(_C // 16):
            dstS[b][pl.ds(i * 16, 16)] = dstv[b][pl.ds(i * 16, 16)]
        issue_scatter(b)
        if issue_idx_next:
            issue_idx(cnum + 3, b)
        if issue_gather_next:
            b2 = (b + 2) % 3
            if not skip_ss_wait:
                wait_scatter(b2)
            wait_idx(b2)
            issue_gather(b2)

    issue_idx(0, 0)
    issue_idx(1, 1)
    issue_idx(2, 2)
    zvec = jnp.zeros((16,), jnp.float32)

    def zrow(i, carry):
        def zcol(j, cc):
            zero_v[i, pl.ds(j * 16, 16)] = zvec
            return cc
        return lax.fori_loop(0, 8, zcol, carry)

    lax.fori_loop(0, _ZR, zrow, 0)
    for z in range(rps // _ZR):
        pltpu.sync_copy(zero_v, acc.at[pl.ds(s * rps + z * _ZR, _ZR)])
    wait_idx(0)
    issue_gather(0)
    wait_idx(1)
    issue_gather(1)
    plsc.subcore_barrier()

    do_chunk(0, 0, skip_ss_wait=True)
    do_chunk(1, 1)
    do_chunk(2, 2)

    def body(g, carry):
        cb = g * 3
        do_chunk(cb + 0, 0)
        do_chunk(cb + 1, 1)
        do_chunk(cb + 2, 2)
        return carry

    lax.fori_loop(1, (_T - 6) // 3 + 1, body, 0)

    do_chunk(_T - 3, 0, issue_idx_next=False)
    do_chunk(_T - 2, 1, issue_idx_next=False, issue_gather_next=False)
    do_chunk(_T - 1, 2, issue_idx_next=False, issue_gather_next=False)
    wait_scatter(0)
    wait_scatter(1)
    wait_scatter(2)

    plsc.subcore_barrier()
    for z in range(rps // _ZR):
        r0 = s * rps + z * _ZR
        pltpu.sync_copy(acc.at[pl.ds(r0, _ZR)], out_hbm.at[c, pl.ds(r0, _ZR)])


def _spmm_sc(src, dst, ew, x):
    n, d = x.shape
    epad = _NW * _T * _C
    pad = epad - src.shape[0]
    src = jnp.concatenate([src, jnp.zeros((pad,), jnp.int32)])
    dst = jnp.concatenate([dst, jnp.full((pad,), _NPAD - 1, jnp.int32)])
    ew = jnp.concatenate([ew, jnp.zeros((pad,), jnp.float32)])
    run = pl.kernel(
        _spmm_body,
        out_type=jax.ShapeDtypeStruct((_NC, _NPAD, d), jnp.float32),
        compiler_params=pltpu.CompilerParams(needs_layout_passes=False),
        mesh=plsc.VectorSubcoreMesh(core_axis_name="c", subcore_axis_name="s"),
        scratch_types=(
            [pltpu.VMEM_SHARED((_NPAD, d), jnp.float32)]
            + [pltpu.VMEM((_C,), jnp.int32) for _ in range(9)]
            + [pltpu.VMEM((_C,), jnp.float32) for _ in range(3)]
            + [pltpu.VMEM((_C, d), jnp.float32) for _ in range(3)]
            + [pltpu.VMEM((_ZR, d), jnp.float32)]
            + [pltpu.SemaphoreType.DMA for _ in range(9)]
        ),
    )
    return run(src, dst, ew, x)


def _finish_body(a_ref, wl_ref, ws_ref, o_ref):
    ax = a_ref[0] + a_ref[1]
    yl = jnp.dot(ax, wl_ref[...], preferred_element_type=jnp.float32)
    ys = jnp.dot(ax, ws_ref[...], preferred_element_type=jnp.float32)
    o_ref[...] = (jnp.where(yl >= 0, yl, 0.01 * yl)
                  + jnp.where(ys >= 0, ys, 0.01 * ys))


def _finish(parts, wl, ws):
    _, n, d = parts.shape
    return pl.pallas_call(
        _finish_body,
        out_shape=jax.ShapeDtypeStruct((n, d), jnp.float32),
    )(parts, wl, ws)


def _gru_cell(Q, Zt, Wz, Uz, bz, Wr, Ur, br, Wh, Uh, bh):
    upd = jax.nn.sigmoid(Wz @ Zt + Uz @ Q + bz)
    rst = jax.nn.sigmoid(Wr @ Zt + Ur @ Q + br)
    hcap = jnp.tanh(Wh @ Zt + Uh @ (rst * Q) + bh)
    return (1.0 - upd) * Q + upd * hcap


def kernel(node_feats, edge_index, edge_weight, mask,
           long_W0, long_p, long_Wz, long_Uz, long_bz, long_Wr, long_Ur,
           long_br, long_Wh, long_Uh, long_bh,
           short_W0, short_p, short_Wz, short_Uz, short_bz, short_Wr,
           short_Ur, short_br, short_Wh, short_Uh, short_bh):
    t_, n, d = node_feats.shape
    short = 3
    nl = jnp.linalg.norm(long_p) + 1e-12
    ns = jnp.linalg.norm(short_p) + 1e-12
    rows_t = np.concatenate([np.arange(t_), np.arange(t_ - short, t_)])
    A = jnp.stack([node_feats[t] @ long_p / nl + mask[t] for t in range(t_)]
                  + [node_feats[t] @ short_p / ns + mask[t]
                     for t in range(t_ - short, t_)])
    vals, idx = lax.top_k(A, d)
    flat = node_feats.reshape(t_ * n, d)
    gathered = flat[(idx + rows_t[:, None] * n).reshape(-1)]
    Zts = (gathered.reshape(-1, d, d)
           * jnp.tanh(vals)[:, :, None]).transpose(0, 2, 1)

    Q = long_W0
    for t in range(t_):
        Q = _gru_cell(Q, Zts[t], long_Wz, long_Uz, long_bz, long_Wr,
                      long_Ur, long_br, long_Wh, long_Uh, long_bh)
    Wl = Q
    Q = short_W0
    for j in range(short):
        Q = _gru_cell(Q, Zts[t_ + j], short_Wz, short_Uz, short_bz,
                      short_Wr, short_Ur, short_br, short_Wh, short_Uh,
                      short_bh)
    Ws = Q

    src = edge_index[-1, 0]
    dst = edge_index[-1, 1]
    parts = _spmm_sc(src, dst, edge_weight[-1], node_feats[-1])
    return _finish(parts, Wl, Ws)[:n]


# speedup vs baseline: 1.3704x; 1.3704x over previous
import jax
import jax.numpy as jnp
import numpy as np
from jax import lax
from jax.experimental import pallas as pl
from jax.experimental.pallas import tpu as pltpu
from jax.experimental.pallas import tpu_sc as plsc

_NC = 2
_NS = 16
_NW = _NC * _NS
_C = 80
_T = 126
_ZR = 128
_NPAD = 10240


def _spmm_body(src_hbm, dst_hbm, w_hbm, x_hbm, out_hbm,
               acc,
               srcv0, srcv1, srcv2, dstv0, dstv1, dstv2,
               dstS0, dstS1, dstS2, wv0, wv1, wv2,
               rows0, rows1, rows2, zero_v,
               sg0, sg1, sg2, ss0, ss1, ss2, si0, si1, si2):
    n = acc.shape[0]
    epw = _T * _C
    c = lax.axis_index("c")
    s = lax.axis_index("s")
    wid = s * _NC + c
    rps = n // _NS

    srcv = [srcv0, srcv1, srcv2]
    dstv = [dstv0, dstv1, dstv2]
    dstS = [dstS0, dstS1, dstS2]
    wv = [wv0, wv1, wv2]
    rows = [rows0, rows1, rows2]
    sg = [sg0, sg1, sg2]
    ss = [ss0, ss1, ss2]
    si = [si0, si1, si2]

    def issue_idx(cnum, b):
        base = wid * epw + cnum * _C
        pltpu.async_copy(src_hbm.at[pl.ds(base, _C)], srcv[b], si[b])
        pltpu.async_copy(dst_hbm.at[pl.ds(base, _C)], dstv[b], si[b])
        pltpu.async_copy(w_hbm.at[pl.ds(base, _C)], wv[b], si[b])

    def wait_idx(b):
        pltpu.make_async_copy(src_hbm.at[pl.ds(0, _C)], srcv[b], si[b]).wait()
        pltpu.make_async_copy(dst_hbm.at[pl.ds(0, _C)], dstv[b], si[b]).wait()
        pltpu.make_async_copy(w_hbm.at[pl.ds(0, _C)], wv[b], si[b]).wait()

    def issue_gather(b):
        pltpu.async_copy(x_hbm.at[srcv[b]], rows[b], sg[b])

    def wait_gather(b):
        pltpu.make_async_copy(x_hbm.at[srcv[b]], rows[b], sg[b]).wait()

    def issue_scatter(b):
        pltpu.async_copy(rows[b], acc.at[dstS[b]], ss[b], add=True)

    def wait_scatter(b):
        pltpu.make_async_copy(rows[b], acc.at[dstS[b]], ss[b]).wait()

    def multiply(b):
        def gloop(g, cc):
            w16 = wv[b][pl.ds(g * 16, 16)]
            for rr in range(16):
                wsp = jnp.broadcast_to(w16[rr], (16,))
                for j in range(8):
                    vec = rows[b][g * 16 + rr, pl.ds(j * 16, 16)]
                    rows[b][g * 16 + rr, pl.ds(j * 16, 16)] = vec * wsp
            return cc
        lax.fori_loop(0, _C // 16, gloop, 0)

    def do_chunk(cnum, b, issue_idx_next=True, issue_gather_next=True,
                 skip_ss_wait=False):
        wait_gather(b)
        multiply(b)
        for i in range(_C // 16):
            dstS[b][pl.ds(i * 16, 16)] = dstv[b][pl.ds(i * 16, 16)]
        issue_scatter(b)
        if issue_idx_next:
            issue_idx(cnum + 3, b)
        if issue_gather_next:
            b2 = (b + 2) % 3
            if not skip_ss_wait:
                wait_scatter(b2)
            wait_idx(b2)
            issue_gather(b2)

    issue_idx(0, 0)
    issue_idx(1, 1)
    issue_idx(2, 2)
    zvec = jnp.zeros((16,), jnp.float32)

    def zrow(i, carry):
        def zcol(j, cc):
            zero_v[i, pl.ds(j * 16, 16)] = zvec
            return cc
        return lax.fori_loop(0, 8, zcol, carry)

    lax.fori_loop(0, _ZR, zrow, 0)
    for z in range(rps // _ZR):
        pltpu.sync_copy(zero_v, acc.at[pl.ds(s * rps + z * _ZR, _ZR)])
    wait_idx(0)
    issue_gather(0)
    wait_idx(1)
    issue_gather(1)
    plsc.subcore_barrier()

    do_chunk(0, 0, skip_ss_wait=True)
    do_chunk(1, 1)
    do_chunk(2, 2)

    def body(g, carry):
        cb = g * 3
        do_chunk(cb + 0, 0)
        do_chunk(cb + 1, 1)
        do_chunk(cb + 2, 2)
        return carry

    lax.fori_loop(1, (_T - 6) // 3 + 1, body, 0)

    do_chunk(_T - 3, 0, issue_idx_next=False)
    do_chunk(_T - 2, 1, issue_idx_next=False, issue_gather_next=False)
    do_chunk(_T - 1, 2, issue_idx_next=False, issue_gather_next=False)
    wait_scatter(0)
    wait_scatter(1)
    wait_scatter(2)

    plsc.subcore_barrier()
    for z in range(rps // _ZR):
        r0 = s * rps + z * _ZR
        pltpu.sync_copy(acc.at[pl.ds(r0, _ZR)], out_hbm.at[c, pl.ds(r0, _ZR)])


def _spmm_sc(src, dst, ew, x):
    n, d = x.shape
    epad = _NW * _T * _C
    pad = epad - src.shape[0]
    src = jnp.concatenate([src, jnp.zeros((pad,), jnp.int32)])
    dst = jnp.concatenate([dst, jnp.full((pad,), _NPAD - 1, jnp.int32)])
    ew = jnp.concatenate([ew, jnp.zeros((pad,), jnp.float32)])
    run = pl.kernel(
        _spmm_body,
        out_type=jax.ShapeDtypeStruct((_NC, _NPAD, d), jnp.float32),
        compiler_params=pltpu.CompilerParams(needs_layout_passes=False),
        mesh=plsc.VectorSubcoreMesh(core_axis_name="c", subcore_axis_name="s"),
        scratch_types=(
            [pltpu.VMEM_SHARED((_NPAD, d), jnp.float32)]
            + [pltpu.VMEM((_C,), jnp.int32) for _ in range(9)]
            + [pltpu.VMEM((_C,), jnp.float32) for _ in range(3)]
            + [pltpu.VMEM((_C, d), jnp.float32) for _ in range(3)]
            + [pltpu.VMEM((_ZR, d), jnp.float32)]
            + [pltpu.SemaphoreType.DMA for _ in range(9)]
        ),
    )
    return run(src, dst, ew, x)


def _finish_body(a_ref, wl_ref, ws_ref, o_ref):
    ax = a_ref[0] + a_ref[1]
    yl = jnp.dot(ax, wl_ref[...], preferred_element_type=jnp.float32)
    ys = jnp.dot(ax, ws_ref[...], preferred_element_type=jnp.float32)
    o_ref[...] = (jnp.where(yl >= 0, yl, 0.01 * yl)
                  + jnp.where(ys >= 0, ys, 0.01 * ys))


def _finish(parts, wl, ws):
    _, n, d = parts.shape
    return pl.pallas_call(
        _finish_body,
        out_shape=jax.ShapeDtypeStruct((n, d), jnp.float32),
    )(parts, wl, ws)


def _evolve(Q, feats, msk, p, Wz, Uz, bz, Wr, Ur, br, Wh, Uh, bh):
    inv = 1.0 / (jnp.linalg.norm(p) + 1e-12)
    for t in range(feats.shape[0]):
        X = feats[t]
        scores = (X @ p) * inv + msk[t]
        vals, idx = lax.top_k(scores, Q.shape[1])
        Zt = (X[idx] * jnp.tanh(vals)[:, None]).T
        upd = jax.nn.sigmoid(Wz @ Zt + Uz @ Q + bz)
        rst = jax.nn.sigmoid(Wr @ Zt + Ur @ Q + br)
        hcap = jnp.tanh(Wh @ Zt + Uh @ (rst * Q) + bh)
        Q = (1.0 - upd) * Q + upd * hcap
    return Q


def kernel(node_feats, edge_index, edge_weight, mask,
           long_W0, long_p, long_Wz, long_Uz, long_bz, long_Wr, long_Ur,
           long_br, long_Wh, long_Uh, long_bh,
           short_W0, short_p, short_Wz, short_Uz, short_bz, short_Wr,
           short_Ur, short_br, short_Wh, short_Uh, short_bh):
    short = 3
    Wl = _evolve(long_W0, node_feats, mask, long_p, long_Wz, long_Uz,
                 long_bz, long_Wr, long_Ur, long_br, long_Wh, long_Uh,
                 long_bh)
    Ws = _evolve(short_W0, node_feats[-short:], mask[-short:], short_p,
                 short_Wz, short_Uz, short_bz, short_Wr, short_Ur, short_br,
                 short_Wh, short_Uh, short_bh)
    src = edge_index[-1, 0]
    dst = edge_index[-1, 1]
    parts = _spmm_sc(src, dst, edge_weight[-1], node_feats[-1])
    return _finish(parts, Wl, Ws)[:node_feats.shape[1]]


# speedup vs baseline: 1.7058x; 1.2448x over previous
import jax
import jax.numpy as jnp
import numpy as np
from jax import lax
from jax.experimental import pallas as pl
from jax.experimental.pallas import tpu as pltpu
from jax.experimental.pallas import tpu_sc as plsc

_NC = 2
_NS = 16
_NW = _NC * _NS
_C = 80
_T = 125
_ZR = 128
_NPAD = 10240


def _spmm_body(soff, doff, woff, xoff,
               ei_hbm, ew_hbm, x_hbm, out_hbm,
               acc,
               srcv0, srcv1, srcv2, dstv0, dstv1, dstv2,
               dstS0, dstS1, dstS2, wv0, wv1, wv2,
               rows0, rows1, rows2, zero_v,
               sg0, sg1, sg2, ss0, ss1, ss2, si0, si1, si2):
    n = acc.shape[0]
    epw = _T * _C
    c = lax.axis_index("c")
    s = lax.axis_index("s")
    wid = s * _NC + c
    rps = n // _NS

    srcv = [srcv0, srcv1, srcv2]
    dstv = [dstv0, dstv1, dstv2]
    dstS = [dstS0, dstS1, dstS2]
    wv = [wv0, wv1, wv2]
    rows = [rows0, rows1, rows2]
    sg = [sg0, sg1, sg2]
    ss = [ss0, ss1, ss2]
    si = [si0, si1, si2]

    def issue_idx(cnum, b):
        cw = jnp.where(cnum >= _T, 0, cnum)
        base = wid * epw + cw * _C
        pltpu.async_copy(ei_hbm.at[pl.ds(soff + base, _C)], srcv[b], si[b])
        pltpu.async_copy(ei_hbm.at[pl.ds(doff + base, _C)], dstv[b], si[b])
        pltpu.async_copy(ew_hbm.at[pl.ds(woff + base, _C)], wv[b], si[b])

    def wait_idx(b):
        pltpu.make_async_copy(ei_hbm.at[pl.ds(0, _C)], srcv[b], si[b]).wait()
        pltpu.make_async_copy(ei_hbm.at[pl.ds(0, _C)], dstv[b], si[b]).wait()
        pltpu.make_async_copy(ew_hbm.at[pl.ds(0, _C)], wv[b], si[b]).wait()

    def issue_gather(b):
        for i in range(_C // 16):
            srcv[b][pl.ds(i * 16, 16)] = srcv[b][pl.ds(i * 16, 16)] + xoff
        pltpu.async_copy(x_hbm.at[srcv[b]], rows[b], sg[b])

    def wait_gather(b):
        pltpu.make_async_copy(x_hbm.at[srcv[b]], rows[b], sg[b]).wait()

    def issue_scatter(b):
        pltpu.async_copy(rows[b], acc.at[dstS[b]], ss[b], add=True)

    def wait_scatter(b):
        pltpu.make_async_copy(rows[b], acc.at[dstS[b]], ss[b]).wait()

    def multiply(b):
        def gloop(g, cc):
            w16 = wv[b][pl.ds(g * 16, 16)]
            for rr in range(16):
                wsp = jnp.broadcast_to(w16[rr], (16,))
                for j in range(8):
                    vec = rows[b][g * 16 + rr, pl.ds(j * 16, 16)]
                    rows[b][g * 16 + rr, pl.ds(j * 16, 16)] = vec * wsp
            return cc
        lax.fori_loop(0, _C // 16, gloop, 0)

    def do_chunk(cnum, b, issue_idx_next=True, issue_gather_next=True,
                 skip_ss_wait=False):
        wait_gather(b)
        multiply(b)
        for i in range(_C // 16):
            dstS[b][pl.ds(i * 16, 16)] = dstv[b][pl.ds(i * 16, 16)]
        issue_scatter(b)
        if issue_idx_next:
            issue_idx(cnum + 3, b)
        if issue_gather_next:
            b2 = (b + 2) % 3
            if not skip_ss_wait:
                wait_scatter(b2)
            wait_idx(b2)
            issue_gather(b2)

    issue_idx(0, 0)
    issue_idx(1, 1)
    issue_idx(2, 2)
    zvec = jnp.zeros((16,), jnp.float32)

    def zrow(i, carry):
        def zcol(j, cc):
            zero_v[i, pl.ds(j * 16, 16)] = zvec
            return cc
        return lax.fori_loop(0, 8, zcol, carry)

    lax.fori_loop(0, _ZR, zrow, 0)
    for z in range(rps // _ZR):
        pltpu.sync_copy(zero_v, acc.at[pl.ds(s * rps + z * _ZR, _ZR)])
    wait_idx(0)
    issue_gather(0)
    wait_idx(1)
    issue_gather(1)
    plsc.subcore_barrier()

    do_chunk(0, 0, skip_ss_wait=True)
    do_chunk(1, 1)
    do_chunk(2, 2)

    def body(g, carry):
        cb = g * 3
        do_chunk(cb + 0, 0)
        do_chunk(cb + 1, 1)
        do_chunk(cb + 2, 2)
        return carry

    lax.fori_loop(1, (_T - 5) // 3 + 1, body, 0)

    do_chunk(_T - 2, 0, issue_idx_next=False, issue_gather_next=False)
    do_chunk(_T - 1, 1, issue_idx_next=False, issue_gather_next=False)
    wait_scatter(2)
    wait_scatter(0)
    wait_scatter(1)
    wait_idx(2)

    plsc.subcore_barrier()
    for z in range(rps // _ZR):
        r0 = s * rps + z * _ZR
        pltpu.sync_copy(acc.at[pl.ds(r0, _ZR)], out_hbm.at[c, pl.ds(r0, _ZR)])


def _spmm_sc(edge_index, edge_weight, node_feats):
    t_, n, d = node_feats.shape
    e = edge_index.shape[2]
    eflat = edge_index.reshape(-1)
    wflat = edge_weight.reshape(-1)
    xflat = node_feats.reshape(t_ * n, d)
    soff = 2 * (t_ - 1) * e
    doff = soff + e
    woff = (t_ - 1) * e
    xoff = (t_ - 1) * n
    import functools
    run = pl.kernel(
        functools.partial(_spmm_body, soff, doff, woff, xoff),
        out_type=jax.ShapeDtypeStruct((_NC, _NPAD, d), jnp.float32),
        compiler_params=pltpu.CompilerParams(needs_layout_passes=False),
        mesh=plsc.VectorSubcoreMesh(core_axis_name="c", subcore_axis_name="s"),
        scratch_types=(
            [pltpu.VMEM_SHARED((_NPAD, d), jnp.float32)]
            + [pltpu.VMEM((_C,), jnp.int32) for _ in range(9)]
            + [pltpu.VMEM((_C,), jnp.float32) for _ in range(3)]
            + [pltpu.VMEM((_C, d), jnp.float32) for _ in range(3)]
            + [pltpu.VMEM((_ZR, d), jnp.float32)]
            + [pltpu.SemaphoreType.DMA for _ in range(9)]
        ),
    )
    return run(eflat, wflat, xflat)


def _finish_body(a_ref, wl_ref, ws_ref, o_ref):
    ax = a_ref[0] + a_ref[1]
    yl = jnp.dot(ax, wl_ref[...], preferred_element_type=jnp.float32)
    ys = jnp.dot(ax, ws_ref[...], preferred_element_type=jnp.float32)
    o_ref[...] = (jnp.where(yl >= 0, yl, 0.01 * yl)
                  + jnp.where(ys >= 0, ys, 0.01 * ys))


def _finish(parts, wl, ws):
    _, n, d = parts.shape
    return pl.pallas_call(
        _finish_body,
        out_shape=jax.ShapeDtypeStruct((n, d), jnp.float32),
    )(parts, wl, ws)


def _evolve(Q, feats, msk, p, Wz, Uz, bz, Wr, Ur, br, Wh, Uh, bh):
    inv = 1.0 / (jnp.linalg.norm(p) + 1e-12)
    for t in range(feats.shape[0]):
        X = feats[t]
        scores = (X @ p) * inv + msk[t]
        vals, idx = lax.top_k(scores, Q.shape[1])
        Zt = (X[idx] * jnp.tanh(vals)[:, None]).T
        upd = jax.nn.sigmoid(Wz @ Zt + Uz @ Q + bz)
        rst = jax.nn.sigmoid(Wr @ Zt + Ur @ Q + br)
        hcap = jnp.tanh(Wh @ Zt + Uh @ (rst * Q) + bh)
        Q = (1.0 - upd) * Q + upd * hcap
    return Q


def kernel(node_feats, edge_index, edge_weight, mask,
           long_W0, long_p, long_Wz, long_Uz, long_bz, long_Wr, long_Ur,
           long_br, long_Wh, long_Uh, long_bh,
           short_W0, short_p, short_Wz, short_Uz, short_bz, short_Wr,
           short_Ur, short_br, short_Wh, short_Uh, short_bh):
    short = 3
    Wl = _evolve(long_W0, node_feats, mask, long_p, long_Wz, long_Uz,
                 long_bz, long_Wr, long_Ur, long_br, long_Wh, long_Uh,
                 long_bh)
    Ws = _evolve(short_W0, node_feats[-short:], mask[-short:], short_p,
                 short_Wz, short_Uz, short_bz, short_Wr, short_Ur, short_br,
                 short_Wh, short_Uh, short_bh)
    parts = _spmm_sc(edge_index, edge_weight, node_feats)
    return _finish(parts, Wl, Ws)[:node_feats.shape[1]]
